# Initial kernel scaffold; baseline (speedup 1.0000x reference)
import jax
import jax.numpy as jnp
from jax.experimental import pallas as pl


def kernel(x, causal_mask, attention_mask, positions, ln1_g, ln1_b, ln2_g, ln2_b, Wrp, Wr, Wq, Wk, Wv, Wo, Wrouter, W1, W2, W3):
    raise NotImplementedError("write your pallas kernel here")


# speedup vs baseline: 1.4492x; 1.4492x over previous
import jax
import jax.numpy as jnp
from jax import lax
from jax.experimental import pallas as pl
from jax.experimental.pallas import tpu as pltpu

B, T = 2, 2048
D, H, KH = 768, 12, 6
DH = 64
E, TOPK, F = 8, 2, 512
NEG = -1e30
TB = 512
NTB = (B * T) // TB
CH = 512

_f32 = jnp.float32


def _ln_body(x, g, b):
    mu = jnp.mean(x, axis=-1, keepdims=True)
    xc = x - mu
    var = jnp.mean(xc * xc, axis=-1, keepdims=True)
    return xc * lax.rsqrt(var + 1e-5) * g + b


def _rot(t):
    half = t.shape[-1] // 2
    return jnp.concatenate([-t[:, half:], t[:, :half]], axis=-1)


def _a1_body(x_ref, g_ref, b_ref, wrpt_ref, cos_ref, sin_ref, h_ref, pooled_ref):
    x = x_ref[0]
    h = _ln_body(x, g_ref[0], b_ref[0])
    h_ref[0] = h
    xr = jnp.dot(h, wrpt_ref[...], preferred_element_type=_f32)
    xr = xr * cos_ref[...] + _rot(xr) * sin_ref[...]
    pooled_ref[0, 0, :] = jnp.sum(xr, axis=0) * (1.0 / T)


def _a2_body(pooled_ref, wrt_ref, rw_ref, ti_ref):
    logits = jnp.dot(pooled_ref[...], wrt_ref[...], preferred_element_type=_f32)
    it = lax.broadcasted_iota(jnp.int32, (B, H), 1)
    vs, ids = [], []
    l = logits
    for _ in range(KH):
        m = jnp.max(l, axis=1, keepdims=True)
        i = jnp.min(jnp.where(l == m, it, H), axis=1, keepdims=True)
        vs.append(m)
        ids.append(i)
        l = jnp.where(it == i, NEG, l)
    tv = jnp.concatenate(vs, axis=1)
    ex = jnp.exp(tv - tv[:, :1])
    w = ex / jnp.sum(ex, axis=1, keepdims=True)
    it128 = lax.broadcasted_iota(jnp.int32, (B, 128), 1)
    rw = jnp.zeros((B, 128), _f32)
    ti = jnp.zeros((B, 128), jnp.int32)
    for kk in range(KH):
        rw = jnp.where(it128 == kk, w[:, kk:kk + 1], rw)
        ti = jnp.where(it128 == kk, ids[kk], ti)
    rw_ref[...] = rw
    ti_ref[...] = ti


def _b_body(tif_ref, rwf_ref, h_ref, wq_ref, wk_ref, wv_ref, wo_ref,
            cos_ref, sin_ref, out_ref):
    b = pl.program_id(0)
    kk = pl.program_id(1)
    h = h_ref[0]
    cos = cos_ref[...]
    sin = sin_ref[...]
    q = jnp.dot(h, wq_ref[0], preferred_element_type=_f32)
    k = jnp.dot(h, wk_ref[0], preferred_element_type=_f32)
    v = jnp.dot(h, wv_ref[0], preferred_element_type=_f32)
    q = q * cos + _rot(q) * sin
    k = k * cos + _rot(k) * sin
    wgt = rwf_ref[b * KH + kk]

    @pl.when(kk == 0)
    def _():
        out_ref[0] = jnp.zeros((T, D), _f32)

    scale = DH ** -0.5
    wo = wo_ref[0]
    for i in range(T // CH):
        qc = q[i * CH:(i + 1) * CH] * scale
        s = lax.dot_general(qc, k, (((1,), (1,)), ((), ())),
                            preferred_element_type=_f32)
        rows = lax.broadcasted_iota(jnp.int32, (CH, T), 0) + i * CH
        cols = lax.broadcasted_iota(jnp.int32, (CH, T), 1)
        s = jnp.where(cols > rows, NEG, s)
        m = jnp.max(s, axis=1, keepdims=True)
        p = jnp.exp(s - m)
        p = p / jnp.sum(p, axis=1, keepdims=True)
        ctx = jnp.dot(p, v, preferred_element_type=_f32)
        oph = jnp.dot(ctx, wo, preferred_element_type=_f32)
        out_ref[0, i * CH:(i + 1) * CH, :] += oph * wgt


def _c0_body(x_ref, a_ref, g_ref, b_ref, wrt_ref, x1_ref, h2_ref, fw_ref):
    x1 = x_ref[...] + a_ref[...]
    x1_ref[...] = x1
    h2 = _ln_body(x1, g_ref[0], b_ref[0])
    h2_ref[...] = h2
    rl = jnp.dot(h2, wrt_ref[...], preferred_element_type=_f32)
    it = lax.broadcasted_iota(jnp.int32, (TB, E), 1)
    m1 = jnp.max(rl, axis=1, keepdims=True)
    i1 = jnp.min(jnp.where(rl == m1, it, E), axis=1, keepdims=True)
    rl2 = jnp.where(it == i1, NEG, rl)
    m2 = jnp.max(rl2, axis=1, keepdims=True)
    i2 = jnp.min(jnp.where(rl2 == m2, it, E), axis=1, keepdims=True)
    w1 = 1.0 / (1.0 + jnp.exp(m2 - m1))
    w2 = 1.0 - w1
    fw = jnp.where(it == i1, w1, 0.0) + jnp.where(it == i2, w2, 0.0)
    fw_ref[...] = fw


def _c1_body(h2_ref, w1_ref, w3_ref, w2_ref, fw_ref, x1_ref, out_ref):
    e = pl.program_id(1)

    @pl.when(e == 0)
    def _():
        out_ref[...] = x1_ref[...]

    h2 = h2_ref[...]
    h1 = jnp.dot(h2, w1_ref[0], preferred_element_type=_f32)
    h3 = jnp.dot(h2, w3_ref[0], preferred_element_type=_f32)
    he = h1 * (1.0 / (1.0 + jnp.exp(-h1))) * h3
    eo = jnp.dot(he, w2_ref[0], preferred_element_type=_f32)
    it = lax.broadcasted_iota(jnp.int32, (TB, E), 1)
    wcol = jnp.sum(jnp.where(it == e, fw_ref[...], 0.0), axis=1, keepdims=True)
    out_ref[...] += eo * wcol


def kernel(x, causal_mask, attention_mask, positions, ln1_g, ln1_b, ln2_g, ln2_b,
           Wrp, Wr, Wq, Wk, Wv, Wo, Wrouter, W1, W2, W3):
    half = DH // 2
    inv_freq = 1.0 / (10000.0 ** (jnp.arange(half, dtype=_f32) * 2.0 / DH))
    ang = positions.astype(_f32)[:, None] * inv_freq[None, :]
    cos = jnp.concatenate([jnp.cos(ang), jnp.cos(ang)], axis=-1)
    sin = jnp.concatenate([jnp.sin(ang), jnp.sin(ang)], axis=-1)

    g1 = ln1_g.reshape(1, D)
    b1 = ln1_b.reshape(1, D)
    g2 = ln2_g.reshape(1, D)
    b2 = ln2_b.reshape(1, D)

    h, pooled = pl.pallas_call(
        _a1_body,
        grid=(B,),
        in_specs=[
            pl.BlockSpec((1, T, D), lambda b: (b, 0, 0)),
            pl.BlockSpec((1, D), lambda b: (0, 0)),
            pl.BlockSpec((1, D), lambda b: (0, 0)),
            pl.BlockSpec((D, DH), lambda b: (0, 0)),
            pl.BlockSpec((T, DH), lambda b: (0, 0)),
            pl.BlockSpec((T, DH), lambda b: (0, 0)),
        ],
        out_specs=[
            pl.BlockSpec((1, T, D), lambda b: (b, 0, 0)),
            pl.BlockSpec((1, 1, DH), lambda b: (b, 0, 0)),
        ],
        out_shape=[
            jax.ShapeDtypeStruct((B, T, D), _f32),
            jax.ShapeDtypeStruct((B, 1, DH), _f32),
        ],
    )(x, g1, b1, Wrp.T, cos, sin)

    rw_pad, ti_pad = pl.pallas_call(
        _a2_body,
        out_shape=[
            jax.ShapeDtypeStruct((B, 128), _f32),
            jax.ShapeDtypeStruct((B, 128), jnp.int32),
        ],
    )(pooled.reshape(B, DH), Wr.T)

    tif = ti_pad[:, :KH].reshape(B * KH)
    rwf = rw_pad[:, :KH].reshape(B * KH)

    attn = pl.pallas_call(
        _b_body,
        grid_spec=pltpu.PrefetchScalarGridSpec(
            num_scalar_prefetch=2,
            grid=(B, KH),
            in_specs=[
                pl.BlockSpec((1, T, D), lambda b, k, tif, rwf: (b, 0, 0)),
                pl.BlockSpec((1, D, DH), lambda b, k, tif, rwf: (tif[b * KH + k], 0, 0)),
                pl.BlockSpec((1, D, DH), lambda b, k, tif, rwf: (tif[b * KH + k], 0, 0)),
                pl.BlockSpec((1, D, DH), lambda b, k, tif, rwf: (tif[b * KH + k], 0, 0)),
                pl.BlockSpec((1, DH, D), lambda b, k, tif, rwf: (tif[b * KH + k], 0, 0)),
                pl.BlockSpec((T, DH), lambda b, k, tif, rwf: (0, 0)),
                pl.BlockSpec((T, DH), lambda b, k, tif, rwf: (0, 0)),
            ],
            out_specs=pl.BlockSpec((1, T, D), lambda b, k, tif, rwf: (b, 0, 0)),
        ),
        out_shape=jax.ShapeDtypeStruct((B, T, D), _f32),
        compiler_params=pltpu.CompilerParams(
            dimension_semantics=("arbitrary", "arbitrary")),
    )(tif, rwf, h, Wq, Wk, Wv, Wo, cos, sin)

    x2 = x.reshape(B * T, D)
    a2 = attn.reshape(B * T, D)
    x1, h2, fw = pl.pallas_call(
        _c0_body,
        grid=(NTB,),
        in_specs=[
            pl.BlockSpec((TB, D), lambda t: (t, 0)),
            pl.BlockSpec((TB, D), lambda t: (t, 0)),
            pl.BlockSpec((1, D), lambda t: (0, 0)),
            pl.BlockSpec((1, D), lambda t: (0, 0)),
            pl.BlockSpec((D, E), lambda t: (0, 0)),
        ],
        out_specs=[
            pl.BlockSpec((TB, D), lambda t: (t, 0)),
            pl.BlockSpec((TB, D), lambda t: (t, 0)),
            pl.BlockSpec((TB, E), lambda t: (t, 0)),
        ],
        out_shape=[
            jax.ShapeDtypeStruct((B * T, D), _f32),
            jax.ShapeDtypeStruct((B * T, D), _f32),
            jax.ShapeDtypeStruct((B * T, E), _f32),
        ],
    )(x2, a2, g2, b2, Wrouter.T)

    out = pl.pallas_call(
        _c1_body,
        grid=(NTB, E),
        in_specs=[
            pl.BlockSpec((TB, D), lambda t, e: (t, 0)),
            pl.BlockSpec((1, D, F), lambda t, e: (e, 0, 0)),
            pl.BlockSpec((1, D, F), lambda t, e: (e, 0, 0)),
            pl.BlockSpec((1, F, D), lambda t, e: (e, 0, 0)),
            pl.BlockSpec((TB, E), lambda t, e: (t, 0)),
            pl.BlockSpec((TB, D), lambda t, e: (t, 0)),
        ],
        out_specs=pl.BlockSpec((TB, D), lambda t, e: (t, 0)),
        out_shape=jax.ShapeDtypeStruct((B * T, D), _f32),
        compiler_params=pltpu.CompilerParams(
            dimension_semantics=("arbitrary", "arbitrary")),
    )(h2, W1, W3, W2, fw, x1)

    return out.reshape(B, T, D)
